# baseline (speedup 1.0000x reference)
import jax
import jax.numpy as jnp
from jax.experimental import pallas as pl
from jax.experimental.pallas import tpu as pltpu

_B, _C, _H, _W = 4, 192, 384, 384
_J, _MH, _MW = 8, 96, 96
_FH, _FW = _H // _MH, _W // _MW
_CC = 24


def _sums_body(x_ref, seg_ref, out_ref, y_ref):
    pw = (jax.lax.broadcasted_iota(jnp.int32, (_W, _MW), 0) // _FW
          == jax.lax.broadcasted_iota(jnp.int32, (_W, _MW), 1)).astype(jnp.float32)
    y = jax.lax.dot_general(
        x_ref[0], pw, (((1,), (0,)), ((), ())), preferred_element_type=jnp.float32
    )
    y_ref[...] = y.reshape(_CC, _H, _MW)
    y2 = y_ref[...].reshape(_CC, _H * _MW)
    seg = seg_ref[0]
    m = jnp.where(seg != 0, 1.0, 0.0)
    mb = jnp.broadcast_to(
        m[:, None, :, :], (_MH, _FH, _MW, _J)
    ).reshape(_H * _MW, _J)
    out_ref[0] = jax.lax.dot_general(
        y2, mb, (((1,), (0,)), ((), ())), preferred_element_type=jnp.float32
    )


def _finish_body(sums_ref, seg_ref, wt_ref, b_ref, out_ref):
    seg = seg_ref[...]
    area = jnp.sum(jnp.where(seg != 0, 1.0, 0.0), axis=(2, 3)) * (_FH * _FW)
    for j in range(_J):
        s = sums_ref[:, j, :]
        a = area[:, j]
        feat = s / jnp.maximum(a, 1.0)[:, None]
        h = jnp.maximum(feat, 0.0)
        o = (
            jax.lax.dot_general(
                h, wt_ref[j], (((1,), (0,)), ((), ())),
                preferred_element_type=jnp.float32,
            )
            + b_ref[j][None, :]
        )
        o = jnp.where((a > 0)[:, None], o, 0.0)
        out_ref[:, j, :] = o


@jax.jit
def kernel(x, segmap_attentions, W, b):
    x3 = x.reshape(_B, _C * _H, _W)
    seg_t = jnp.transpose(segmap_attentions, (0, 2, 3, 1))
    sums_cj = pl.pallas_call(
        _sums_body,
        grid=(_B, _C // _CC),
        in_specs=[
            pl.BlockSpec((1, _CC * _H, _W), lambda b_, t: (b_, t, 0)),
            pl.BlockSpec((1, _MH, _MW, _J), lambda b_, t: (b_, 0, 0, 0)),
        ],
        out_specs=pl.BlockSpec((1, _CC, _J), lambda b_, t: (b_, t, 0)),
        out_shape=jax.ShapeDtypeStruct((_B, _C, _J), jnp.float32),
        scratch_shapes=[pltpu.VMEM((_CC, _H, _MW), jnp.float32)],
        compiler_params=pltpu.CompilerParams(
            dimension_semantics=("parallel", "arbitrary"),
        ),
    )(x3, seg_t)

    sums_jc = jnp.transpose(sums_cj, (0, 2, 1))
    wt = jnp.transpose(W, (0, 2, 1))
    out = pl.pallas_call(
        _finish_body,
        out_shape=jax.ShapeDtypeStruct((_B, _J, _C), jnp.float32),
    )(sums_jc, segmap_attentions, wt, b)
    return out


# speedup vs baseline: 1.0414x; 1.0414x over previous
import jax
import jax.numpy as jnp
from jax.experimental import pallas as pl
from jax.experimental.pallas import tpu as pltpu

_B, _C, _H, _W = 4, 192, 384, 384
_J, _MH, _MW = 8, 96, 96
_FH, _FW = _H // _MH, _W // _MW
_CC = 32


def _sums_body(x_ref, seg_ref, out_ref, y_ref):
    pw = (jax.lax.broadcasted_iota(jnp.int32, (_W, _MW), 0) // _FW
          == jax.lax.broadcasted_iota(jnp.int32, (_W, _MW), 1)).astype(jnp.float32)
    y = jax.lax.dot_general(
        x_ref[0], pw, (((1,), (0,)), ((), ())), preferred_element_type=jnp.float32
    )
    y_ref[...] = y.reshape(_CC, _H, _MW)
    y2 = y_ref[...].reshape(_CC, _H * _MW)
    seg = seg_ref[0]
    m = jnp.where(seg != 0, 1.0, 0.0)
    mb = jnp.broadcast_to(
        m[:, None, :, :], (_MH, _FH, _MW, _J)
    ).reshape(_H * _MW, _J)
    out_ref[0] = jax.lax.dot_general(
        y2, mb, (((1,), (0,)), ((), ())), preferred_element_type=jnp.float32
    )


def _finish_body(sums_ref, seg_ref, wt_ref, b_ref, out_ref):
    seg = seg_ref[...]
    area = jnp.sum(jnp.where(seg != 0, 1.0, 0.0), axis=(2, 3)) * (_FH * _FW)
    for j in range(_J):
        s = sums_ref[:, j, :]
        a = area[:, j]
        feat = s / jnp.maximum(a, 1.0)[:, None]
        h = jnp.maximum(feat, 0.0)
        o = (
            jax.lax.dot_general(
                h, wt_ref[j], (((1,), (0,)), ((), ())),
                preferred_element_type=jnp.float32,
            )
            + b_ref[j][None, :]
        )
        o = jnp.where((a > 0)[:, None], o, 0.0)
        out_ref[:, j, :] = o


@jax.jit
def kernel(x, segmap_attentions, W, b):
    x3 = x.reshape(_B, _C * _H, _W)
    seg_t = jnp.transpose(segmap_attentions, (0, 2, 3, 1))
    sums_cj = pl.pallas_call(
        _sums_body,
        grid=(_B, _C // _CC),
        in_specs=[
            pl.BlockSpec((1, _CC * _H, _W), lambda b_, t: (b_, t, 0)),
            pl.BlockSpec((1, _MH, _MW, _J), lambda b_, t: (b_, 0, 0, 0)),
        ],
        out_specs=pl.BlockSpec((1, _CC, _J), lambda b_, t: (b_, t, 0)),
        out_shape=jax.ShapeDtypeStruct((_B, _C, _J), jnp.float32),
        scratch_shapes=[pltpu.VMEM((_CC, _H, _MW), jnp.float32)],
        compiler_params=pltpu.CompilerParams(
            dimension_semantics=("parallel", "arbitrary"),
        ),
    )(x3, seg_t)

    sums_jc = jnp.transpose(sums_cj, (0, 2, 1))
    wt = jnp.transpose(W, (0, 2, 1))
    out = pl.pallas_call(
        _finish_body,
        out_shape=jax.ShapeDtypeStruct((_B, _J, _C), jnp.float32),
    )(sums_jc, segmap_attentions, wt, b)
    return out
